# baseline (speedup 1.0000x reference)
import functools

import jax
import jax.numpy as jnp
from jax import lax
from jax.experimental import pallas as pl
from jax.experimental.pallas import tpu as pltpu
from jax.experimental.pallas import tpu_sc as plsc

_NC = 2
_NS = 16
_NW = _NC * _NS


def _fcg_body(rows_per_w, rows_last, ch, nchunk,
              ci_hbm, c1_hbm, c2_hbm, c3_hbm, o_hbm,
              ci_v, c1_v, c2_v, c3_v, o1_v, o2_v, o3_v):
    m = ci_hbm.shape[0]
    wid = lax.axis_index("s") * _NC + lax.axis_index("c")
    base = wid * rows_per_w
    nrows = jnp.where(wid == _NW - 1, rows_last, rows_per_w)
    last_start = base + nrows - ch

    n_vregs = ch // 16

    def chunk_body(i, carry):
        start = jnp.minimum(base + i * ch, last_start)
        pltpu.sync_copy(ci_hbm.at[pl.ds(start, ch)], ci_v)
        pltpu.sync_copy(c1_hbm.at[pl.ds(start, ch)], c1_v)
        pltpu.sync_copy(c2_hbm.at[pl.ds(start, ch)], c2_v)
        pltpu.sync_copy(c3_hbm.at[pl.ds(start, ch)], c3_v)

        def vreg_body(g, c2_):
            s = pl.ds(g * 16, 16)
            civ = ci_v[s]
            o1_v[s] = (c1_v[s] << 1) + (civ & 1)
            o2_v[s] = (c2_v[s] << 1) + ((civ >> 1) & 1)
            o3_v[s] = (c3_v[s] << 1) + ((civ >> 2) & 1)
            return c2_

        lax.fori_loop(0, n_vregs, vreg_body, 0)
        pltpu.sync_copy(o1_v, o_hbm.at[pl.ds(start, ch)])
        pltpu.sync_copy(o2_v, o_hbm.at[pl.ds(m + start, ch)])
        pltpu.sync_copy(o3_v, o_hbm.at[pl.ds(2 * m + start, ch)])
        return carry

    lax.fori_loop(0, nchunk, chunk_body, 0)


def _feat_copy_body(cols_per_w, cols_last, tail_start, tail, ch, nchunk,
                    src, dst):
    wid = lax.axis_index("s") * _NC + lax.axis_index("c")
    base = wid * cols_per_w
    ncols = jnp.where(wid == _NW - 1, cols_last, cols_per_w)
    last_start = base + ncols - ch

    def chunk_body(i, carry):
        start = jnp.minimum(base + i * ch, last_start)
        pltpu.sync_copy(src.at[:, pl.ds(start, ch)],
                        dst.at[:, pl.ds(start, ch)])
        return carry

    lax.fori_loop(0, nchunk, chunk_body, 0)

    if tail:
        @pl.when(wid == _NW - 1)
        def _():
            pltpu.sync_copy(src.at[:, pl.ds(tail_start, tail)],
                            dst.at[:, pl.ds(tail_start, tail)])


def kernel(parent_coords, child_indices, parent_features):
    m = parent_coords.shape[0]
    assert m % 16 == 0 and m // _NW >= 16
    rows_per_w = (m // _NW) // 16 * 16
    rows_last = m - (_NW - 1) * rows_per_w
    ch = min(8192, rows_per_w) // 16 * 16
    nchunk = -(-rows_last // ch)

    ci = child_indices.astype(jnp.int32)

    body = functools.partial(_fcg_body, rows_per_w, rows_last, ch, nchunk)
    o3m = pl.kernel(
        body,
        out_type=jax.ShapeDtypeStruct((3 * m,), jnp.int32),
        mesh=plsc.VectorSubcoreMesh(core_axis_name="c", subcore_axis_name="s"),
        scratch_types=[pltpu.VMEM((ch,), jnp.int32) for _ in range(7)],
    )(ci, parent_coords[:, 1], parent_coords[:, 2], parent_coords[:, 3])
    nblk = m // 128
    blk_per_w = nblk // _NW
    cols_per_w = blk_per_w * 128
    cols_last = (nblk - (_NW - 1) * blk_per_w) * 128
    tail_start = nblk * 128
    tail = m - tail_start
    fch = cols_per_w
    fchunk = -(-cols_last // fch)
    fbody = functools.partial(_feat_copy_body, cols_per_w, cols_last,
                              tail_start, tail, fch, fchunk)
    pfT = pl.kernel(
        fbody,
        out_type=jax.ShapeDtypeStruct(parent_features.shape[::-1],
                                      parent_features.dtype),
        mesh=plsc.VectorSubcoreMesh(core_axis_name="c", subcore_axis_name="s"),
    )(parent_features.T)
    pf = pfT.T
    out = jnp.stack(
        [parent_coords[:, 0], o3m[:m], o3m[m:2 * m], o3m[2 * m:]], axis=1)
    return out, pf


# speedup vs baseline: 46.3849x; 46.3849x over previous
import functools

import jax
import jax.numpy as jnp
from jax import lax
from jax.experimental import pallas as pl
from jax.experimental.pallas import tpu as pltpu
from jax.experimental.pallas import tpu_sc as plsc

_NC = 2
_NS = 16
_NW = _NC * _NS


def _fcg_body(rows_per_w, rows_last, ch, nchunk,
              ci_hbm, c1_hbm, c2_hbm, c3_hbm, o1_hbm, o2_hbm, o3_hbm,
              ci_v, c1_v, c2_v, c3_v, o1_v, o2_v, o3_v):
    wid = lax.axis_index("s") * _NC + lax.axis_index("c")
    base = wid * rows_per_w
    nrows = jnp.where(wid == _NW - 1, rows_last, rows_per_w)
    last_start = base + nrows - ch

    n_vregs = ch // 16

    def chunk_body(i, carry):
        start = jnp.minimum(base + i * ch, last_start)
        pltpu.sync_copy(ci_hbm.at[pl.ds(start, ch)], ci_v)
        pltpu.sync_copy(c1_hbm.at[pl.ds(start, ch)], c1_v)
        pltpu.sync_copy(c2_hbm.at[pl.ds(start, ch)], c2_v)
        pltpu.sync_copy(c3_hbm.at[pl.ds(start, ch)], c3_v)

        def vreg_body(g, c2_):
            s = pl.ds(g * 16, 16)
            civ = ci_v[s]
            o1_v[s] = (c1_v[s] << 1) + (civ & 1)
            o2_v[s] = (c2_v[s] << 1) + ((civ >> 1) & 1)
            o3_v[s] = (c3_v[s] << 1) + ((civ >> 2) & 1)
            return c2_

        lax.fori_loop(0, n_vregs, vreg_body, 0)
        pltpu.sync_copy(o1_v, o1_hbm.at[pl.ds(start, ch)])
        pltpu.sync_copy(o2_v, o2_hbm.at[pl.ds(start, ch)])
        pltpu.sync_copy(o3_v, o3_hbm.at[pl.ds(start, ch)])
        return carry

    lax.fori_loop(0, nchunk, chunk_body, 0)


def kernel(parent_coords, child_indices, parent_features):
    m = parent_coords.shape[0]
    assert m % 16 == 0 and m // _NW >= 16
    rows_per_w = (m // _NW) // 16 * 16
    rows_last = m - (_NW - 1) * rows_per_w
    ch = min(8192, rows_per_w) // 16 * 16
    nchunk = -(-rows_last // ch)

    ci = child_indices.astype(jnp.int32)

    body = functools.partial(_fcg_body, rows_per_w, rows_last, ch, nchunk)
    o1, o2, o3 = pl.kernel(
        body,
        out_type=[jax.ShapeDtypeStruct((m,), jnp.int32) for _ in range(3)],
        mesh=plsc.VectorSubcoreMesh(core_axis_name="c", subcore_axis_name="s"),
        scratch_types=[pltpu.VMEM((ch,), jnp.int32) for _ in range(7)],
    )(ci, parent_coords[:, 1], parent_coords[:, 2], parent_coords[:, 3])
    one = parent_features[0, 0] * 0.0 + 1.0
    pf = parent_features * one
    o1, o2, o3, pf = lax.optimization_barrier((o1, o2, o3, pf))
    out = jnp.stack([parent_coords[:, 0], o1, o2, o3], axis=1)
    return out, pf
